# baseline (speedup 1.0000x reference)
import jax
import jax.numpy as jnp
from jax.experimental import pallas as pl
from jax.experimental.pallas import tpu as pltpu

NS = 4
C = 96
H = 224
W = 224
BH = 32
NBLK = H // BH

_f32 = jnp.float32
_bf16 = jnp.bfloat16


def _gumbel_cm(ch_mask):
    u = jax.random.uniform(jax.random.key(1234), ch_mask.shape,
                           minval=1e-6, maxval=1.0 - 1e-6, dtype=_f32)
    g = -jnp.log(-jnp.log(u))
    return jax.nn.softmax((ch_mask + g) / 1.0, axis=3)


def _dot(a, b):
    return jax.lax.dot_general(a, b, (((1,), (0,)), ((), ())),
                               preferred_element_type=_f32)


def _wshift3(fea, rows):
    z = jnp.zeros((rows, 1, C), _bf16)
    right = jnp.concatenate([z, fea[:, 0 : W - 1, :]], axis=1)
    left = jnp.concatenate([fea[:, 1:W, :], z], axis=1)
    return right, fea, left


def _fused_kernel(xp_ref, spa_ref, w0_ref, wmid_ref, wf_ref, cv_ref, bc_ref,
                  out_ref, slx_ref, sl0_ref, sl1_ref):
    sl2_ref = slx_ref
    blk = pl.program_id(0)
    r0 = blk * BH

    x40 = xp_ref[pl.ds(r0, BH + 8), :, :]
    for dw in range(3):
        slx_ref[dw, :, :, :] = x40[:, dw : dw + W, :]

    r = BH + 6
    acc = jnp.zeros((r * W, C), _f32)
    for dh in range(3):
        for dw in range(3):
            xs = slx_ref[dw, dh : dh + r, :, :].reshape(r * W, C)
            acc += _dot(xs, w0_ref[dh * 3 + dw])
    spa = spa_ref[pl.ds(r0 + 1, r), :, :].astype(_f32)
    u = cv_ref[0, 0, :]
    v = cv_ref[0, 1, :]
    b0 = cv_ref[0, 2, :]
    pre = (acc.reshape(r, W, C) + b0) * (u * spa + v)
    gid = jax.lax.broadcasted_iota(jnp.int32, (r, 1, 1), 0) + (r0 - 3)
    m = ((gid >= 0) & (gid < H)).astype(_f32)
    fea = jnp.maximum(pre * m, 0.0).astype(_bf16)
    for dw, sv in zip(range(3), _wshift3(fea, r)):
        sl0_ref[dw, :, :, :] = sv

    slabs = (sl0_ref, sl1_ref, sl2_ref)
    fea3 = None
    for s in range(1, NS):
        rin = r
        r = rin - 2
        src = slabs[s - 1]
        acc = jnp.zeros((r * W, 2 * 128), _f32)
        for dh in range(3):
            for dw in range(3):
                xs = src[dw, dh : dh + r, :, :].reshape(r * W, C)
                acc += _dot(xs, wmid_ref[s - 1, dh * 3 + dw])
        del src
        cc = jax.lax.slice(acc, (0, 0), (r * W, C)).reshape(r, W, C)
        dd = jax.lax.slice(acc, (0, 128), (r * W, 128 + C)).reshape(r, W, C)
        spa = spa_ref[pl.ds(r0 + 1 + s, r), :, :].astype(_f32)
        a1 = cv_ref[s, 0, :]
        tc = cv_ref[s, 1, :]
        ts = cv_ref[s, 2, :]
        t = dd * a1
        pre = spa * (cc + ts - t) + t + tc
        if s < 3:
            gid = jax.lax.broadcasted_iota(jnp.int32, (r, 1, 1), 0) + (r0 - 3 + s)
            m = ((gid >= 0) & (gid < H)).astype(_f32)
            pre = pre * m
        fea = jnp.maximum(pre, 0.0).astype(_bf16)
        if s < 3:
            for dw, sv in zip(range(3), _wshift3(fea, r)):
                slabs[s][dw, 0:r, :, :] = sv
        else:
            fea3 = fea

    acc = jnp.zeros((BH * W, C), _f32)
    acc += _dot(sl0_ref[1, 3 : 3 + BH, :, :].reshape(BH * W, C), wf_ref[0])
    acc += _dot(sl1_ref[1, 2 : 2 + BH, :, :].reshape(BH * W, C), wf_ref[1])
    acc += _dot(sl2_ref[1, 1 : 1 + BH, :, :].reshape(BH * W, C), wf_ref[2])
    acc += _dot(fea3.reshape(BH * W, C), wf_ref[3])
    out_ref[...] = (acc + bc_ref[0, :]).reshape(BH, W, C).astype(_bf16)


def kernel(x0, x1, ch_mask, w0, b0, w1, b1, w2, b2, w3, b3, wc, bc):
    cm = _gumbel_cm(ch_mask)
    spa = jnp.transpose(x1[0], (1, 2, 0)).astype(_bf16)
    spa4 = jnp.pad(spa, ((4, 4), (0, 0), (0, 0)))
    x = jnp.transpose(x0[0], (1, 2, 0)).astype(_bf16)
    xp = jnp.pad(x, ((4, 4), (1, 1), (0, 0)))

    w0k = jnp.transpose(w0, (2, 3, 1, 0)).reshape(9, C, C).astype(_bf16)

    zrow = [jnp.zeros((C,), _f32)] * 5
    wmids = []
    cvs = [jnp.stack([cm[0, :, 0, 0], cm[0, :, 0, 1], b0] + zrow)]
    ws = (w1, w2, w3)
    bs = (b1, b2, b3)
    for i in range(1, NS):
        wik = jnp.transpose(ws[i - 1], (2, 3, 1, 0))
        d = cm[0, :, i - 1, 1]
        wm = jnp.zeros((9, C, 2 * 128), _f32)
        wm = wm.at[:, :, 0:C].set(wik.reshape(9, C, C))
        wm = wm.at[:, :, 128 : 128 + C].set(
            (wik * d[None, None, :, None]).reshape(9, C, C))
        wmids.append(wm)
        a0 = cm[0, :, i, 0]
        a1 = cm[0, :, i, 1]
        bi = bs[i - 1]
        cvs.append(jnp.stack([a1, bi * a1, bi * (a0 + 1.0)] + zrow))
    wmid = jnp.stack(wmids).astype(_bf16)
    cv = jnp.stack(cvs)
    wf = jnp.transpose(wc.reshape(C, NS, C), (1, 2, 0)).astype(_bf16)
    bcv = jnp.stack([bc] + [jnp.zeros((C,), _f32)] * 7)

    def full(shape):
        return pl.BlockSpec(shape, lambda *_: tuple(0 for _ in shape))

    y = pl.pallas_call(
        _fused_kernel, grid=(NBLK,),
        in_specs=[
            full((H + 8, W + 2, C)),
            full((H + 8, W, 1)),
            full((9, C, C)),
            full((NS - 1, 9, C, 2 * 128)),
            full((NS, C, C)),
            full((NS, 8, C)),
            full((8, C)),
        ],
        out_specs=pl.BlockSpec((BH, W, C), lambda i: (i, 0, 0)),
        out_shape=jax.ShapeDtypeStruct((H, W, C), _bf16),
        scratch_shapes=[
            pltpu.VMEM((3, BH + 8, W, C), _bf16),
            pltpu.VMEM((3, BH + 6, W, C), _bf16),
            pltpu.VMEM((3, BH + 4, W, C), _bf16),
        ],
        compiler_params=pltpu.CompilerParams(
            vmem_limit_bytes=100 * 1024 * 1024),
    )(xp, spa4, w0k, wmid, wf, cv, bcv)
    y = jnp.transpose(y.astype(_f32), (2, 0, 1))[None]
    return y, cm


# speedup vs baseline: 1.4391x; 1.4391x over previous
import jax
import jax.numpy as jnp
from jax.experimental import pallas as pl
from jax.experimental.pallas import tpu as pltpu

NS = 4
C = 96
H = 224
W = 224
BH = 32
NBLK = H // BH

_f32 = jnp.float32
_bf16 = jnp.bfloat16


def _gumbel_cm(ch_mask):
    u = jax.random.uniform(jax.random.key(1234), ch_mask.shape,
                           minval=1e-6, maxval=1.0 - 1e-6, dtype=_f32)
    g = -jnp.log(-jnp.log(u))
    return jax.nn.softmax((ch_mask + g) / 1.0, axis=3)


def _rows8(*vs):
    pad = [jnp.zeros((C,), _f32)] * (8 - len(vs))
    return jnp.stack(list(vs) + pad)


def _dot(a, b):
    return jax.lax.dot_general(a, b, (((1,), (0,)), ((), ())),
                               preferred_element_type=_f32)


def _stage0_kernel(xp_ref, spa_ref, w_ref, cv_ref, out_ref):
    r0 = pl.program_id(0) * BH
    acc = jnp.zeros((BH * W, C), _f32)
    for dh in range(3):
        for dw in range(3):
            xs = xp_ref[pl.ds(r0 + dh, BH), pl.ds(dw, W), :].reshape(BH * W, C)
            acc += _dot(xs, w_ref[dh * 3 + dw])
    spa = spa_ref[...]
    u = cv_ref[0, :]
    v = cv_ref[1, :]
    b = cv_ref[2, :]
    t = acc.reshape(BH, W, C) + b
    fea = t * (u * spa + v)
    out_ref[...] = jnp.maximum(fea, 0.0).astype(_bf16)


def _mid_math(acc, spa, cv_row):
    a1 = cv_row[0, :]
    tc = cv_row[1, :]
    ts = cv_row[2, :]
    cc = jax.lax.slice(acc, (0, 0), (BH * W, C)).reshape(BH, W, C)
    dd = jax.lax.slice(acc, (0, 128), (BH * W, 128 + C)).reshape(BH, W, C)
    t = dd * a1
    fea = spa * (cc + ts - t) + t + tc
    return jnp.maximum(fea, 0.0).astype(_bf16)


def _mid_stage_kernel(xp_ref, spa_ref, wm_ref, cv_ref, out_ref):
    r0 = pl.program_id(0) * BH
    acc = jnp.zeros((BH * W, 2 * 128), _f32)
    for dh in range(3):
        for dw in range(3):
            xs = xp_ref[pl.ds(r0 + dh, BH), pl.ds(dw, W), :].reshape(BH * W, C)
            acc += _dot(xs, wm_ref[dh * 3 + dw])
    out_ref[...] = _mid_math(acc, spa_ref[...], cv_ref)


def _last_stage_kernel(xp_ref, spa_ref, wm_ref, cv_ref, f0_ref, f1_ref,
                       f2_ref, wf_ref, bc_ref, out_ref):
    r0 = pl.program_id(0) * BH
    acc = jnp.zeros((BH * W, 2 * 128), _f32)
    for dh in range(3):
        for dw in range(3):
            xs = xp_ref[pl.ds(r0 + dh, BH), pl.ds(dw, W), :].reshape(BH * W, C)
            acc += _dot(xs, wm_ref[dh * 3 + dw])
    fea3 = _mid_math(acc, spa_ref[...], cv_ref)
    acc = jnp.zeros((BH * W, C), _f32)
    for i, f in enumerate((f0_ref, f1_ref, f2_ref)):
        acc += _dot(f[...].reshape(BH * W, C), wf_ref[i])
    acc += _dot(fea3.reshape(BH * W, C), wf_ref[3])
    out_ref[...] = (acc + bc_ref[0, :]).reshape(BH, W, C)


_GRID = (NBLK,)
_XSPEC = pl.BlockSpec((H + 2, W + 2, C), lambda i: (0, 0, 0))
_SPASPEC = pl.BlockSpec((BH, W, 1), lambda i: (i, 0, 0))
_W9SPEC = pl.BlockSpec((9, C, C), lambda i: (0, 0, 0))
_WMSPEC = pl.BlockSpec((9, C, 2 * 128), lambda i: (0, 0, 0))
_W4SPEC = pl.BlockSpec((NS, C, C), lambda i: (0, 0, 0))
_CVSPEC = pl.BlockSpec((8, C), lambda i: (0, 0))
_OSPEC = pl.BlockSpec((BH, W, C), lambda i: (i, 0, 0))
_OSHAPE = jax.ShapeDtypeStruct((H, W, C), _bf16)
_YSHAPE = jax.ShapeDtypeStruct((H, W, C), _f32)
_CP = pltpu.CompilerParams(vmem_limit_bytes=100 * 1024 * 1024)


def kernel(x0, x1, ch_mask, w0, b0, w1, b1, w2, b2, w3, b3, wc, bc):
    cm = _gumbel_cm(ch_mask)
    spa = jnp.transpose(x1[0], (1, 2, 0))
    x = jnp.transpose(x0[0], (1, 2, 0)).astype(_bf16)
    xp = jnp.pad(x, ((1, 1), (1, 1), (0, 0)))

    w0k = jnp.transpose(w0, (2, 3, 1, 0)).reshape(9, C, C).astype(_bf16)
    cv0 = _rows8(cm[0, :, 0, 0], cm[0, :, 0, 1], b0)
    fea = pl.pallas_call(
        _stage0_kernel, grid=_GRID,
        in_specs=[_XSPEC, _SPASPEC, _W9SPEC, _CVSPEC],
        out_specs=_OSPEC, out_shape=_OSHAPE, compiler_params=_CP,
    )(xp, spa, w0k, cv0)
    outs = [fea]

    ws = (w1, w2, w3)
    bs = (b1, b2, b3)
    w4 = jnp.transpose(wc.reshape(C, NS, C), (1, 2, 0)).astype(_bf16)
    bcv = _rows8(bc)
    for i in range(1, NS):
        wik = jnp.transpose(ws[i - 1], (2, 3, 1, 0))
        d = cm[0, :, i - 1, 1]
        wm = jnp.zeros((9, C, 2 * 128), _f32)
        wm = wm.at[:, :, 0:C].set(wik.reshape(9, C, C))
        wm = wm.at[:, :, 128 : 128 + C].set(
            (wik * d[None, None, :, None]).reshape(9, C, C))
        wm = wm.astype(_bf16)
        a0 = cm[0, :, i, 0]
        a1 = cm[0, :, i, 1]
        bi = bs[i - 1]
        cv = _rows8(a1, bi * a1, bi * (a0 + 1.0))
        xpi = jnp.pad(fea, ((1, 1), (1, 1), (0, 0)))
        if i < NS - 1:
            fea = pl.pallas_call(
                _mid_stage_kernel, grid=_GRID,
                in_specs=[_XSPEC, _SPASPEC, _WMSPEC, _CVSPEC],
                out_specs=_OSPEC, out_shape=_OSHAPE, compiler_params=_CP,
            )(xpi, spa, wm, cv)
            outs.append(fea)
        else:
            y = pl.pallas_call(
                _last_stage_kernel, grid=_GRID,
                in_specs=[_XSPEC, _SPASPEC, _WMSPEC, _CVSPEC] +
                         [_OSPEC] * 3 + [_W4SPEC, _CVSPEC],
                out_specs=_OSPEC, out_shape=_YSHAPE, compiler_params=_CP,
            )(xpi, spa, wm, cv, *outs, w4, bcv)
    y = jnp.transpose(y, (2, 0, 1))[None]
    return y, cm


# speedup vs baseline: 1.6257x; 1.1296x over previous
import jax
import jax.numpy as jnp
from jax.experimental import pallas as pl
from jax.experimental.pallas import tpu as pltpu

NS = 4
C = 96
H = 224
W = 224
BH = 32
NBLK = H // BH

_f32 = jnp.float32
_bf16 = jnp.bfloat16


def _gumbel_cm(ch_mask):
    u = jax.random.uniform(jax.random.key(1234), ch_mask.shape,
                           minval=1e-6, maxval=1.0 - 1e-6, dtype=_f32)
    g = -jnp.log(-jnp.log(u))
    return jax.nn.softmax((ch_mask + g) / 1.0, axis=3)


def _rows8(*vs):
    pad = [jnp.zeros((C,), _f32)] * (8 - len(vs))
    return jnp.stack(list(vs) + pad)


def _dot(a, b):
    return jax.lax.dot_general(a, b, (((1,), (0,)), ((), ())),
                               preferred_element_type=_f32)


def _fill_slab(xp_ref, sl_ref, r0):
    for dw in range(3):
        sl_ref[dw, :, :, :] = xp_ref[pl.ds(r0, BH + 2), pl.ds(dw, W), :]


def _stage0_kernel(xp_ref, spa_ref, w_ref, cv_ref, out_ref, sl_ref):
    r0 = pl.program_id(0) * BH
    _fill_slab(xp_ref, sl_ref, r0)
    acc = jnp.zeros((BH * W, C), _f32)
    for dh in range(3):
        for dw in range(3):
            xs = sl_ref[dw, dh : dh + BH, :, :].reshape(BH * W, C)
            acc += _dot(xs, w_ref[dh * 3 + dw])
    spa = spa_ref[...]
    u = cv_ref[0, :]
    v = cv_ref[1, :]
    b = cv_ref[2, :]
    t = acc.reshape(BH, W, C) + b
    fea = t * (u * spa + v)
    out_ref[...] = jnp.maximum(fea, 0.0).astype(_bf16)


def _mid_math(acc, spa, cv_row):
    a1 = cv_row[0, :]
    tc = cv_row[1, :]
    ts = cv_row[2, :]
    cc = jax.lax.slice(acc, (0, 0), (BH * W, C)).reshape(BH, W, C)
    dd = jax.lax.slice(acc, (0, 128), (BH * W, 128 + C)).reshape(BH, W, C)
    t = dd * a1
    fea = spa * (cc + ts - t) + t + tc
    return jnp.maximum(fea, 0.0).astype(_bf16)


def _mid_stage_kernel(xp_ref, spa_ref, wm_ref, cv_ref, out_ref, sl_ref):
    r0 = pl.program_id(0) * BH
    _fill_slab(xp_ref, sl_ref, r0)
    acc = jnp.zeros((BH * W, 2 * 128), _f32)
    for dh in range(3):
        for dw in range(3):
            xs = sl_ref[dw, dh : dh + BH, :, :].reshape(BH * W, C)
            acc += _dot(xs, wm_ref[dh * 3 + dw])
    out_ref[...] = _mid_math(acc, spa_ref[...], cv_ref)


def _last_stage_kernel(xp_ref, spa_ref, wm_ref, cv_ref, f0_ref, f1_ref,
                       f2_ref, wf_ref, bc_ref, out_ref, sl_ref):
    r0 = pl.program_id(0) * BH
    _fill_slab(xp_ref, sl_ref, r0)
    acc = jnp.zeros((BH * W, 2 * 128), _f32)
    for dh in range(3):
        for dw in range(3):
            xs = sl_ref[dw, dh : dh + BH, :, :].reshape(BH * W, C)
            acc += _dot(xs, wm_ref[dh * 3 + dw])
    fea3 = _mid_math(acc, spa_ref[...], cv_ref)
    acc = jnp.zeros((BH * W, C), _f32)
    for i, f in enumerate((f0_ref, f1_ref, f2_ref)):
        acc += _dot(f[...].reshape(BH * W, C), wf_ref[i])
    acc += _dot(fea3.reshape(BH * W, C), wf_ref[3])
    out_ref[...] = (acc + bc_ref[0, :]).reshape(BH, W, C)


_GRID = (NBLK,)
_XSPEC = pl.BlockSpec((H + 2, W + 2, C), lambda i: (0, 0, 0))
_SPASPEC = pl.BlockSpec((BH, W, 1), lambda i: (i, 0, 0))
_W9SPEC = pl.BlockSpec((9, C, C), lambda i: (0, 0, 0))
_WMSPEC = pl.BlockSpec((9, C, 2 * 128), lambda i: (0, 0, 0))
_W4SPEC = pl.BlockSpec((NS, C, C), lambda i: (0, 0, 0))
_CVSPEC = pl.BlockSpec((8, C), lambda i: (0, 0))
_OSPEC = pl.BlockSpec((BH, W, C), lambda i: (i, 0, 0))
_OSHAPE = jax.ShapeDtypeStruct((H, W, C), _bf16)
_YSHAPE = jax.ShapeDtypeStruct((H, W, C), _f32)
_CP = pltpu.CompilerParams(vmem_limit_bytes=100 * 1024 * 1024)
_SLAB = [pltpu.VMEM((3, BH + 2, W, C), _bf16)]


def kernel(x0, x1, ch_mask, w0, b0, w1, b1, w2, b2, w3, b3, wc, bc):
    cm = _gumbel_cm(ch_mask)
    spa = jnp.transpose(x1[0], (1, 2, 0))
    x = jnp.transpose(x0[0], (1, 2, 0)).astype(_bf16)
    xp = jnp.pad(x, ((1, 1), (1, 1), (0, 0)))

    w0k = jnp.transpose(w0, (2, 3, 1, 0)).reshape(9, C, C).astype(_bf16)
    cv0 = _rows8(cm[0, :, 0, 0], cm[0, :, 0, 1], b0)
    fea = pl.pallas_call(
        _stage0_kernel, grid=_GRID,
        in_specs=[_XSPEC, _SPASPEC, _W9SPEC, _CVSPEC],
        out_specs=_OSPEC, out_shape=_OSHAPE, compiler_params=_CP,
        scratch_shapes=_SLAB,
    )(xp, spa, w0k, cv0)
    outs = [fea]

    w4 = jnp.transpose(wc.reshape(C, NS, C), (1, 2, 0)).astype(_bf16)
    bcv = _rows8(bc)
    wall = jnp.stack([w1, w2, w3])
    wallk = jnp.transpose(wall, (0, 3, 4, 2, 1)).reshape(3, 9, C, C)
    dall = jnp.transpose(cm[0, :, 0:3, 1])
    wdall = wallk * dall[:, None, :, None]
    zpad = jnp.zeros((3, 9, C, 128 - C), _f32)
    wmall = jnp.concatenate([wallk, zpad, wdall, zpad], axis=3).astype(_bf16)
    a0all = cm[0, :, 1:NS, 0].T
    a1all = cm[0, :, 1:NS, 1].T
    ball = jnp.stack([b1, b2, b3])
    cvall = jnp.stack(
        [a1all, ball * a1all, ball * (a0all + 1.0)] +
        [jnp.zeros((3, C), _f32)] * 5, axis=1)
    for i in range(1, NS):
        wm = wmall[i - 1]
        cv = cvall[i - 1]
        xpi = jnp.pad(fea, ((1, 1), (1, 1), (0, 0)))
        if i < NS - 1:
            fea = pl.pallas_call(
                _mid_stage_kernel, grid=_GRID,
                in_specs=[_XSPEC, _SPASPEC, _WMSPEC, _CVSPEC],
                out_specs=_OSPEC, out_shape=_OSHAPE, compiler_params=_CP,
                scratch_shapes=_SLAB,
            )(xpi, spa, wm, cv)
            outs.append(fea)
        else:
            y = pl.pallas_call(
                _last_stage_kernel, grid=_GRID,
                in_specs=[_XSPEC, _SPASPEC, _WMSPEC, _CVSPEC] +
                         [_OSPEC] * 3 + [_W4SPEC, _CVSPEC],
                out_specs=_OSPEC, out_shape=_YSHAPE, compiler_params=_CP,
                scratch_shapes=_SLAB,
            )(xpi, spa, wm, cv, *outs, w4, bcv)
    y = jnp.transpose(y, (2, 0, 1))[None]
    return y, cm


# speedup vs baseline: 1.8523x; 1.1394x over previous
import jax
import jax.numpy as jnp
from jax.experimental import pallas as pl
from jax.experimental.pallas import tpu as pltpu

NS = 4
C = 96
H = 224
W = 224
BH = 32
NBLK = H // BH

_f32 = jnp.float32
_bf16 = jnp.bfloat16


def _gumbel_cm(ch_mask):
    u = jax.random.uniform(jax.random.key(1234), ch_mask.shape,
                           minval=1e-6, maxval=1.0 - 1e-6, dtype=_f32)
    g = -jnp.log(-jnp.log(u))
    return jax.nn.softmax((ch_mask + g) / 1.0, axis=3)


def _rows8(*vs):
    pad = [jnp.zeros((C,), _f32)] * (8 - len(vs))
    return jnp.stack(list(vs) + pad)


def _dot(a, b):
    return jax.lax.dot_general(a, b, (((1,), (0,)), ((), ())),
                               preferred_element_type=_f32)


def _wshift(win, dw, rows):
    if dw == 1:
        return win
    z = jnp.zeros((rows, 1, C), _bf16)
    if dw == 0:
        return jnp.concatenate([z, win[:, 0 : W - 1, :]], axis=1)
    return jnp.concatenate([win[:, 1:W, :], z], axis=1)


def _fill_slab(x_ref, sl_ref, blk):
    r0 = blk * BH
    zrow = jnp.zeros((1, W, C), _bf16)

    @pl.when(blk == 0)
    def _():
        win = x_ref[pl.ds(0, BH + 1), :, :]
        for dw in range(3):
            sl_ref[dw, 0:1, :, :] = zrow
            sl_ref[dw, 1 : BH + 2, :, :] = _wshift(win, dw, BH + 1)

    @pl.when(blk == NBLK - 1)
    def _():
        win = x_ref[pl.ds(H - BH - 1, BH + 1), :, :]
        for dw in range(3):
            sl_ref[dw, 0 : BH + 1, :, :] = _wshift(win, dw, BH + 1)
            sl_ref[dw, BH + 1 : BH + 2, :, :] = zrow

    @pl.when((blk > 0) & (blk < NBLK - 1))
    def _():
        win = x_ref[pl.ds(r0 - 1, BH + 2), :, :]
        for dw in range(3):
            sl_ref[dw, :, :, :] = _wshift(win, dw, BH + 2)


def _stage0_kernel(xp_ref, spa_ref, w_ref, cv_ref, out_ref, sl_ref):
    blk = pl.program_id(0)
    r0 = blk * BH
    _fill_slab(xp_ref, sl_ref, blk)
    acc = jnp.zeros((BH * W, C), _f32)
    for dh in range(3):
        for dw in range(3):
            xs = sl_ref[dw, dh : dh + BH, :, :].reshape(BH * W, C)
            acc += _dot(xs, w_ref[dh * 3 + dw])
    spa = spa_ref[...].astype(_f32)
    u = cv_ref[0, :]
    v = cv_ref[1, :]
    b = cv_ref[2, :]
    t = acc.reshape(BH, W, C) + b
    fea = t * (u * spa + v)
    out_ref[...] = jnp.maximum(fea, 0.0).astype(_bf16)


def _mid_math(acc, spa, cv_row):
    spa = spa.astype(_f32)
    a1 = cv_row[0, :]
    tc = cv_row[1, :]
    ts = cv_row[2, :]
    cc = jax.lax.slice(acc, (0, 0), (BH * W, C)).reshape(BH, W, C)
    dd = jax.lax.slice(acc, (0, 128), (BH * W, 128 + C)).reshape(BH, W, C)
    t = dd * a1
    fea = spa * (cc + ts - t) + t + tc
    return jnp.maximum(fea, 0.0).astype(_bf16)


def _mid_stage_kernel(xp_ref, spa_ref, wm_ref, cv_ref, out_ref, sl_ref):
    blk = pl.program_id(0)
    r0 = blk * BH
    _fill_slab(xp_ref, sl_ref, blk)
    acc = jnp.zeros((BH * W, 2 * 128), _f32)
    for dh in range(3):
        for dw in range(3):
            xs = sl_ref[dw, dh : dh + BH, :, :].reshape(BH * W, C)
            acc += _dot(xs, wm_ref[dh * 3 + dw])
    out_ref[...] = _mid_math(acc, spa_ref[...], cv_ref)


def _last_stage_kernel(xp_ref, spa_ref, wm_ref, cv_ref, f0_ref, f1_ref,
                       f2_ref, wf_ref, bc_ref, out_ref, sl_ref):
    blk = pl.program_id(0)
    r0 = blk * BH
    _fill_slab(xp_ref, sl_ref, blk)
    acc = jnp.zeros((BH * W, 2 * 128), _f32)
    for dh in range(3):
        for dw in range(3):
            xs = sl_ref[dw, dh : dh + BH, :, :].reshape(BH * W, C)
            acc += _dot(xs, wm_ref[dh * 3 + dw])
    fea3 = _mid_math(acc, spa_ref[...], cv_ref)
    acc = jnp.zeros((BH * W, C), _f32)
    for i, f in enumerate((f0_ref, f1_ref, f2_ref)):
        acc += _dot(f[...].reshape(BH * W, C), wf_ref[i])
    acc += _dot(fea3.reshape(BH * W, C), wf_ref[3])
    y3 = (acc + bc_ref[0, :]).reshape(BH, W, C)
    out_ref[...] = jnp.transpose(y3, (2, 0, 1))


_GRID = (NBLK,)
_XSPEC = pl.BlockSpec((H, W, C), lambda i: (0, 0, 0))
_SPASPEC = pl.BlockSpec((BH, W, 1), lambda i: (i, 0, 0))
_W9SPEC = pl.BlockSpec((9, C, C), lambda i: (0, 0, 0))
_WMSPEC = pl.BlockSpec((9, C, 2 * 128), lambda i: (0, 0, 0))
_W4SPEC = pl.BlockSpec((NS, C, C), lambda i: (0, 0, 0))
_CVSPEC = pl.BlockSpec((8, C), lambda i: (0, 0))
_OSPEC = pl.BlockSpec((BH, W, C), lambda i: (i, 0, 0))
_YSPEC = pl.BlockSpec((C, BH, W), lambda i: (0, i, 0))
_OSHAPE = jax.ShapeDtypeStruct((H, W, C), _bf16)
_YSHAPE = jax.ShapeDtypeStruct((C, H, W), _f32)
_CP = pltpu.CompilerParams(vmem_limit_bytes=100 * 1024 * 1024)
_SLAB = [pltpu.VMEM((3, BH + 2, W, C), _bf16)]


def kernel(x0, x1, ch_mask, w0, b0, w1, b1, w2, b2, w3, b3, wc, bc):
    cm = _gumbel_cm(ch_mask)
    spa = jnp.transpose(x1[0], (1, 2, 0)).astype(_bf16)
    xp = jnp.transpose(x0[0], (1, 2, 0)).astype(_bf16)

    w0k = jnp.transpose(w0, (2, 3, 1, 0)).reshape(9, C, C).astype(_bf16)
    cv0 = _rows8(cm[0, :, 0, 0], cm[0, :, 0, 1], b0)
    fea = pl.pallas_call(
        _stage0_kernel, grid=_GRID,
        in_specs=[_XSPEC, _SPASPEC, _W9SPEC, _CVSPEC],
        out_specs=_OSPEC, out_shape=_OSHAPE, compiler_params=_CP,
        scratch_shapes=_SLAB,
    )(xp, spa, w0k, cv0)
    outs = [fea]

    w4 = jnp.transpose(wc.reshape(C, NS, C), (1, 2, 0)).astype(_bf16)
    bcv = _rows8(bc)
    wall = jnp.stack([w1, w2, w3])
    wallk = jnp.transpose(wall, (0, 3, 4, 2, 1)).reshape(3, 9, C, C)
    dall = jnp.transpose(cm[0, :, 0:3, 1])
    wdall = wallk * dall[:, None, :, None]
    zpad = jnp.zeros((3, 9, C, 128 - C), _f32)
    wmall = jnp.concatenate([wallk, zpad, wdall, zpad], axis=3).astype(_bf16)
    a0all = cm[0, :, 1:NS, 0].T
    a1all = cm[0, :, 1:NS, 1].T
    ball = jnp.stack([b1, b2, b3])
    cvall = jnp.stack(
        [a1all, ball * a1all, ball * (a0all + 1.0)] +
        [jnp.zeros((3, C), _f32)] * 5, axis=1)
    for i in range(1, NS):
        wm = wmall[i - 1]
        cv = cvall[i - 1]
        xpi = fea
        if i < NS - 1:
            fea = pl.pallas_call(
                _mid_stage_kernel, grid=_GRID,
                in_specs=[_XSPEC, _SPASPEC, _WMSPEC, _CVSPEC],
                out_specs=_OSPEC, out_shape=_OSHAPE, compiler_params=_CP,
                scratch_shapes=_SLAB,
            )(xpi, spa, wm, cv)
            outs.append(fea)
        else:
            y = pl.pallas_call(
                _last_stage_kernel, grid=_GRID,
                in_specs=[_XSPEC, _SPASPEC, _WMSPEC, _CVSPEC] +
                         [_OSPEC] * 3 + [_W4SPEC, _CVSPEC],
                out_specs=_YSPEC, out_shape=_YSHAPE, compiler_params=_CP,
                scratch_shapes=_SLAB,
            )(xpi, spa, wm, cv, *outs, w4, bcv)
    return y[None], cm


# speedup vs baseline: 1.9097x; 1.0310x over previous
import jax
import jax.numpy as jnp
from jax.experimental import pallas as pl
from jax.experimental.pallas import tpu as pltpu

NS = 4
C = 96
H = 224
W = 224
BH = 32
NBLK = H // BH

_f32 = jnp.float32
_bf16 = jnp.bfloat16


def _gumbel_cm(ch_mask):
    u = jax.random.uniform(jax.random.key(1234), ch_mask.shape,
                           minval=1e-6, maxval=1.0 - 1e-6, dtype=_f32)
    g = -jnp.log(-jnp.log(u))
    return jax.nn.softmax((ch_mask + g) / 1.0, axis=3)


def _rows8(*vs):
    pad = [jnp.zeros((C,), _f32)] * (8 - len(vs))
    return jnp.stack(list(vs) + pad)


def _dot(a, b):
    return jax.lax.dot_general(a, b, (((1,), (0,)), ((), ())),
                               preferred_element_type=_f32)


def _wshift(win, dw, rows):
    if dw == 1:
        return win
    z = jnp.zeros((rows, 1, C), _bf16)
    if dw == 0:
        return jnp.concatenate([z, win[:, 0 : W - 1, :]], axis=1)
    return jnp.concatenate([win[:, 1:W, :], z], axis=1)


def _fill_slab(x_ref, sl_ref, blk):
    r0 = blk * BH
    zrow = jnp.zeros((1, W, C), _bf16)

    @pl.when(blk == 0)
    def _():
        win = x_ref[pl.ds(0, BH + 1), :, :]
        for dw in range(3):
            sl_ref[dw, 0:1, :, :] = zrow
            sl_ref[dw, 1 : BH + 2, :, :] = _wshift(win, dw, BH + 1)

    @pl.when(blk == NBLK - 1)
    def _():
        win = x_ref[pl.ds(H - BH - 1, BH + 1), :, :]
        for dw in range(3):
            sl_ref[dw, 0 : BH + 1, :, :] = _wshift(win, dw, BH + 1)
            sl_ref[dw, BH + 1 : BH + 2, :, :] = zrow

    @pl.when((blk > 0) & (blk < NBLK - 1))
    def _():
        win = x_ref[pl.ds(r0 - 1, BH + 2), :, :]
        for dw in range(3):
            sl_ref[dw, :, :, :] = _wshift(win, dw, BH + 2)


def _nchw_win(x_ref, aligned_r, load_rows, off, rows):
    win = x_ref[:, pl.ds(aligned_r, load_rows), :].astype(_bf16)
    win = jnp.transpose(win, (1, 2, 0))
    return win[off : off + rows]


def _fill_slab0(x_ref, sl_ref, blk):
    zrow = jnp.zeros((1, W, C), _bf16)

    @pl.when(blk == 0)
    def _():
        win = _nchw_win(x_ref, 0, BH + 8, 0, BH + 1)
        for dw in range(3):
            sl_ref[dw, 0:1, :, :] = zrow
            sl_ref[dw, 1 : BH + 2, :, :] = _wshift(win, dw, BH + 1)

    @pl.when(blk == NBLK - 1)
    def _():
        win = _nchw_win(x_ref, H - BH - 8, BH + 8, 7, BH + 1)
        for dw in range(3):
            sl_ref[dw, 0 : BH + 1, :, :] = _wshift(win, dw, BH + 1)
            sl_ref[dw, BH + 1 : BH + 2, :, :] = zrow

    @pl.when((blk > 0) & (blk < NBLK - 1))
    def _():
        win = _nchw_win(x_ref, blk * BH - 8, BH + 16, 7, BH + 2)
        for dw in range(3):
            sl_ref[dw, :, :, :] = _wshift(win, dw, BH + 2)


def _stage0_kernel(xp_ref, spa_ref, w_ref, cv_ref, out_ref, sl_ref):
    blk = pl.program_id(0)
    r0 = blk * BH
    _fill_slab0(xp_ref, sl_ref, blk)
    acc = jnp.zeros((BH * W, C), _f32)
    for dh in range(3):
        for dw in range(3):
            xs = sl_ref[dw, dh : dh + BH, :, :].reshape(BH * W, C)
            acc += _dot(xs, w_ref[dh * 3 + dw])
    spa = spa_ref[...].astype(_f32)
    u = cv_ref[0, :]
    v = cv_ref[1, :]
    b = cv_ref[2, :]
    t = acc.reshape(BH, W, C) + b
    fea = t * (u * spa + v)
    out_ref[...] = jnp.maximum(fea, 0.0).astype(_bf16)


def _mid_math(acc, spa, cv_row):
    spa = spa.astype(_f32)
    a1 = cv_row[0, :]
    tc = cv_row[1, :]
    ts = cv_row[2, :]
    cc = jax.lax.slice(acc, (0, 0), (BH * W, C)).reshape(BH, W, C)
    dd = jax.lax.slice(acc, (0, 128), (BH * W, 128 + C)).reshape(BH, W, C)
    t = dd * a1
    fea = spa * (cc + ts - t) + t + tc
    return jnp.maximum(fea, 0.0).astype(_bf16)


def _mid_stage_kernel(xp_ref, spa_ref, wm_ref, cv_ref, out_ref, sl_ref):
    blk = pl.program_id(0)
    r0 = blk * BH
    _fill_slab(xp_ref, sl_ref, blk)
    acc = jnp.zeros((BH * W, 2 * 128), _f32)
    for dh in range(3):
        for dw in range(3):
            xs = sl_ref[dw, dh : dh + BH, :, :].reshape(BH * W, C)
            acc += _dot(xs, wm_ref[dh * 3 + dw])
    out_ref[...] = _mid_math(acc, spa_ref[...], cv_ref)


def _last_stage_kernel(xp_ref, spa_ref, wm_ref, cv_ref, f0_ref, f1_ref,
                       f2_ref, wf_ref, bc_ref, out_ref, sl_ref):
    blk = pl.program_id(0)
    r0 = blk * BH
    _fill_slab(xp_ref, sl_ref, blk)
    acc = jnp.zeros((BH * W, 2 * 128), _f32)
    for dh in range(3):
        for dw in range(3):
            xs = sl_ref[dw, dh : dh + BH, :, :].reshape(BH * W, C)
            acc += _dot(xs, wm_ref[dh * 3 + dw])
    fea3 = _mid_math(acc, spa_ref[...], cv_ref)
    acc = jnp.zeros((BH * W, C), _f32)
    for i, f in enumerate((f0_ref, f1_ref, f2_ref)):
        acc += _dot(f[...].reshape(BH * W, C), wf_ref[i])
    acc += _dot(fea3.reshape(BH * W, C), wf_ref[3])
    y3 = (acc + bc_ref[0, :]).reshape(BH, W, C)
    out_ref[...] = jnp.transpose(y3, (2, 0, 1))


_GRID = (NBLK,)
_XSPEC = pl.BlockSpec((H, W, C), lambda i: (0, 0, 0))
_X0SPEC = pl.BlockSpec((C, H, W), lambda i: (0, 0, 0))
_SPASPEC = pl.BlockSpec((BH, W, 1), lambda i: (i, 0, 0))
_W9SPEC = pl.BlockSpec((9, C, C), lambda i: (0, 0, 0))
_WMSPEC = pl.BlockSpec((9, C, 2 * 128), lambda i: (0, 0, 0))
_W4SPEC = pl.BlockSpec((NS, C, C), lambda i: (0, 0, 0))
_CVSPEC = pl.BlockSpec((8, C), lambda i: (0, 0))
_OSPEC = pl.BlockSpec((BH, W, C), lambda i: (i, 0, 0))
_YSPEC = pl.BlockSpec((C, BH, W), lambda i: (0, i, 0))
_OSHAPE = jax.ShapeDtypeStruct((H, W, C), _bf16)
_YSHAPE = jax.ShapeDtypeStruct((C, H, W), _f32)
_CP = pltpu.CompilerParams(vmem_limit_bytes=100 * 1024 * 1024)
_SLAB = [pltpu.VMEM((3, BH + 2, W, C), _bf16)]


def kernel(x0, x1, ch_mask, w0, b0, w1, b1, w2, b2, w3, b3, wc, bc):
    cm = _gumbel_cm(ch_mask)
    spa = jnp.transpose(x1[0], (1, 2, 0)).astype(_bf16)
    xp = x0[0]

    w0k = jnp.transpose(w0, (2, 3, 1, 0)).reshape(9, C, C).astype(_bf16)
    cv0 = _rows8(cm[0, :, 0, 0], cm[0, :, 0, 1], b0)
    fea = pl.pallas_call(
        _stage0_kernel, grid=_GRID,
        in_specs=[_X0SPEC, _SPASPEC, _W9SPEC, _CVSPEC],
        out_specs=_OSPEC, out_shape=_OSHAPE, compiler_params=_CP,
        scratch_shapes=_SLAB,
    )(xp, spa, w0k, cv0)
    outs = [fea]

    w4 = jnp.transpose(wc.reshape(C, NS, C), (1, 2, 0)).astype(_bf16)
    bcv = _rows8(bc)
    wall = jnp.stack([w1, w2, w3])
    wallk = jnp.transpose(wall, (0, 3, 4, 2, 1)).reshape(3, 9, C, C)
    dall = jnp.transpose(cm[0, :, 0:3, 1])
    wdall = wallk * dall[:, None, :, None]
    zpad = jnp.zeros((3, 9, C, 128 - C), _f32)
    wmall = jnp.concatenate([wallk, zpad, wdall, zpad], axis=3).astype(_bf16)
    a0all = cm[0, :, 1:NS, 0].T
    a1all = cm[0, :, 1:NS, 1].T
    ball = jnp.stack([b1, b2, b3])
    cvall = jnp.stack(
        [a1all, ball * a1all, ball * (a0all + 1.0)] +
        [jnp.zeros((3, C), _f32)] * 5, axis=1)
    for i in range(1, NS):
        wm = wmall[i - 1]
        cv = cvall[i - 1]
        xpi = fea
        if i < NS - 1:
            fea = pl.pallas_call(
                _mid_stage_kernel, grid=_GRID,
                in_specs=[_XSPEC, _SPASPEC, _WMSPEC, _CVSPEC],
                out_specs=_OSPEC, out_shape=_OSHAPE, compiler_params=_CP,
                scratch_shapes=_SLAB,
            )(xpi, spa, wm, cv)
            outs.append(fea)
        else:
            y = pl.pallas_call(
                _last_stage_kernel, grid=_GRID,
                in_specs=[_XSPEC, _SPASPEC, _WMSPEC, _CVSPEC] +
                         [_OSPEC] * 3 + [_W4SPEC, _CVSPEC],
                out_specs=_YSPEC, out_shape=_YSHAPE, compiler_params=_CP,
                scratch_shapes=_SLAB,
            )(xpi, spa, wm, cv, *outs, w4, bcv)
    return y[None], cm


# speedup vs baseline: 1.9927x; 1.0435x over previous
import jax
import jax.numpy as jnp
from jax.experimental import pallas as pl
from jax.experimental.pallas import tpu as pltpu

NS = 4
C = 96
H = 224
W = 224
BH = 32
NBLK = H // BH

_f32 = jnp.float32
_bf16 = jnp.bfloat16


def _gumbel_cm(ch_mask):
    u = jax.random.uniform(jax.random.key(1234), ch_mask.shape,
                           minval=1e-6, maxval=1.0 - 1e-6, dtype=_f32)
    g = -jnp.log(-jnp.log(u))
    return jax.nn.softmax((ch_mask + g) / 1.0, axis=3)


def _rows8(*vs):
    pad = [jnp.zeros((C,), _f32)] * (8 - len(vs))
    return jnp.stack(list(vs) + pad)


def _dot(a, b):
    return jax.lax.dot_general(a, b, (((1,), (0,)), ((), ())),
                               preferred_element_type=_f32)


def _wshift(win, dw, rows):
    if dw == 1:
        return win
    z = jnp.zeros((rows, 1, C), _bf16)
    if dw == 0:
        return jnp.concatenate([z, win[:, 0 : W - 1, :]], axis=1)
    return jnp.concatenate([win[:, 1:W, :], z], axis=1)


def _fill_slab(x_ref, sl_ref, blk):
    r0 = blk * BH
    zrow = jnp.zeros((1, W, C), _bf16)

    @pl.when(blk == 0)
    def _():
        win = x_ref[pl.ds(0, BH + 1), :, :]
        for dw in range(3):
            sl_ref[dw, 0:1, :, :] = zrow
            sl_ref[dw, 1 : BH + 2, :, :] = _wshift(win, dw, BH + 1)

    @pl.when(blk == NBLK - 1)
    def _():
        win = x_ref[pl.ds(H - BH - 1, BH + 1), :, :]
        for dw in range(3):
            sl_ref[dw, 0 : BH + 1, :, :] = _wshift(win, dw, BH + 1)
            sl_ref[dw, BH + 1 : BH + 2, :, :] = zrow

    @pl.when((blk > 0) & (blk < NBLK - 1))
    def _():
        win = x_ref[pl.ds(r0 - 1, BH + 2), :, :]
        for dw in range(3):
            sl_ref[dw, :, :, :] = _wshift(win, dw, BH + 2)


def _nchw_win(x_ref, aligned_r, load_rows, off, rows):
    win = x_ref[:, pl.ds(aligned_r, load_rows), :].astype(_bf16)
    win = jnp.transpose(win, (1, 2, 0))
    return win[off : off + rows]


def _fill_slab0(x_ref, sl_ref, blk):
    zrow = jnp.zeros((1, W, C), _bf16)

    @pl.when(blk == 0)
    def _():
        win = _nchw_win(x_ref, 0, BH + 8, 0, BH + 1)
        for dw in range(3):
            sl_ref[dw, 0:1, :, :] = zrow
            sl_ref[dw, 1 : BH + 2, :, :] = _wshift(win, dw, BH + 1)

    @pl.when(blk == NBLK - 1)
    def _():
        win = _nchw_win(x_ref, H - BH - 8, BH + 8, 7, BH + 1)
        for dw in range(3):
            sl_ref[dw, 0 : BH + 1, :, :] = _wshift(win, dw, BH + 1)
            sl_ref[dw, BH + 1 : BH + 2, :, :] = zrow

    @pl.when((blk > 0) & (blk < NBLK - 1))
    def _():
        win = _nchw_win(x_ref, blk * BH - 8, BH + 16, 7, BH + 2)
        for dw in range(3):
            sl_ref[dw, :, :, :] = _wshift(win, dw, BH + 2)


def _stage0_kernel(xp_ref, spa_ref, w_ref, cv_ref, out_ref, sl_ref):
    blk = pl.program_id(0)
    r0 = blk * BH
    _fill_slab0(xp_ref, sl_ref, blk)
    acc = jnp.zeros((BH * W, C), _f32)
    for dh in range(3):
        for dw in range(3):
            xs = sl_ref[dw, dh : dh + BH, :, :].reshape(BH * W, C)
            acc += _dot(xs, w_ref[dh * 3 + dw])
    spa = spa_ref[...].astype(_f32)[:, :, None]
    u = cv_ref[0, :]
    v = cv_ref[1, :]
    b = cv_ref[2, :]
    t = acc.reshape(BH, W, C) + b
    fea = t * (u * spa + v)
    out_ref[...] = jnp.maximum(fea, 0.0).astype(_bf16)


def _mid_math(acc, spa, cv_row):
    spa = spa.astype(_f32)[:, :, None]
    a1 = cv_row[0, :]
    tc = cv_row[1, :]
    ts = cv_row[2, :]
    cc = jax.lax.slice(acc, (0, 0), (BH * W, C)).reshape(BH, W, C)
    dd = jax.lax.slice(acc, (0, 128), (BH * W, 128 + C)).reshape(BH, W, C)
    t = dd * a1
    fea = spa * (cc + ts - t) + t + tc
    return jnp.maximum(fea, 0.0).astype(_bf16)


def _mid_stage_kernel(xp_ref, spa_ref, wm_ref, cv_ref, out_ref, sl_ref):
    blk = pl.program_id(0)
    r0 = blk * BH
    _fill_slab(xp_ref, sl_ref, blk)
    acc = jnp.zeros((BH * W, 2 * 128), _f32)
    for dh in range(3):
        for dw in range(3):
            xs = sl_ref[dw, dh : dh + BH, :, :].reshape(BH * W, C)
            acc += _dot(xs, wm_ref[dh * 3 + dw])
    out_ref[...] = _mid_math(acc, spa_ref[...], cv_ref)


def _last_stage_kernel(xp_ref, spa_ref, wm_ref, cv_ref, f0_ref, f1_ref,
                       f2_ref, wf_ref, bc_ref, out_ref, sl_ref):
    blk = pl.program_id(0)
    r0 = blk * BH
    _fill_slab(xp_ref, sl_ref, blk)
    acc = jnp.zeros((BH * W, 2 * 128), _f32)
    for dh in range(3):
        for dw in range(3):
            xs = sl_ref[dw, dh : dh + BH, :, :].reshape(BH * W, C)
            acc += _dot(xs, wm_ref[dh * 3 + dw])
    fea3 = _mid_math(acc, spa_ref[...], cv_ref)
    acc = jnp.zeros((BH * W, C), _f32)
    for i, f in enumerate((f0_ref, f1_ref, f2_ref)):
        acc += _dot(f[...].reshape(BH * W, C), wf_ref[i])
    acc += _dot(fea3.reshape(BH * W, C), wf_ref[3])
    y3 = (acc + bc_ref[0, :]).reshape(BH, W, C)
    out_ref[...] = jnp.transpose(y3, (2, 0, 1))


_GRID = (NBLK,)
_XSPEC = pl.BlockSpec((H, W, C), lambda i: (0, 0, 0))
_X0SPEC = pl.BlockSpec((C, H, W), lambda i: (0, 0, 0))
_SPASPEC = pl.BlockSpec((BH, W), lambda i: (i, 0))
_W9SPEC = pl.BlockSpec((9, C, C), lambda i: (0, 0, 0))
_WMSPEC = pl.BlockSpec((9, C, 2 * 128), lambda i: (0, 0, 0))
_W4SPEC = pl.BlockSpec((NS, C, C), lambda i: (0, 0, 0))
_CVSPEC = pl.BlockSpec((8, C), lambda i: (0, 0))
_OSPEC = pl.BlockSpec((BH, W, C), lambda i: (i, 0, 0))
_YSPEC = pl.BlockSpec((C, BH, W), lambda i: (0, i, 0))
_OSHAPE = jax.ShapeDtypeStruct((H, W, C), _bf16)
_YSHAPE = jax.ShapeDtypeStruct((C, H, W), _f32)
_CP = pltpu.CompilerParams(vmem_limit_bytes=100 * 1024 * 1024)
_SLAB = [pltpu.VMEM((3, BH + 2, W, C), _bf16)]


def kernel(x0, x1, ch_mask, w0, b0, w1, b1, w2, b2, w3, b3, wc, bc):
    cm = _gumbel_cm(ch_mask)
    spa = x1[0, 0].astype(_bf16)
    xp = x0[0]

    w0k = jnp.transpose(w0, (2, 3, 1, 0)).reshape(9, C, C).astype(_bf16)
    cv0 = _rows8(cm[0, :, 0, 0], cm[0, :, 0, 1], b0)
    fea = pl.pallas_call(
        _stage0_kernel, grid=_GRID,
        in_specs=[_X0SPEC, _SPASPEC, _W9SPEC, _CVSPEC],
        out_specs=_OSPEC, out_shape=_OSHAPE, compiler_params=_CP,
        scratch_shapes=_SLAB,
    )(xp, spa, w0k, cv0)
    outs = [fea]

    w4 = jnp.transpose(wc.reshape(C, NS, C), (1, 2, 0)).astype(_bf16)
    bcv = _rows8(bc)
    wall = jnp.stack([w1, w2, w3])
    wallk = jnp.transpose(wall, (0, 3, 4, 2, 1)).reshape(3, 9, C, C)
    dall = jnp.transpose(cm[0, :, 0:3, 1])
    wdall = wallk * dall[:, None, :, None]
    zpad = jnp.zeros((3, 9, C, 128 - C), _f32)
    wmall = jnp.concatenate([wallk, zpad, wdall, zpad], axis=3).astype(_bf16)
    a0all = cm[0, :, 1:NS, 0].T
    a1all = cm[0, :, 1:NS, 1].T
    ball = jnp.stack([b1, b2, b3])
    cvall = jnp.stack(
        [a1all, ball * a1all, ball * (a0all + 1.0)] +
        [jnp.zeros((3, C), _f32)] * 5, axis=1)
    for i in range(1, NS):
        wm = wmall[i - 1]
        cv = cvall[i - 1]
        xpi = fea
        if i < NS - 1:
            fea = pl.pallas_call(
                _mid_stage_kernel, grid=_GRID,
                in_specs=[_XSPEC, _SPASPEC, _WMSPEC, _CVSPEC],
                out_specs=_OSPEC, out_shape=_OSHAPE, compiler_params=_CP,
                scratch_shapes=_SLAB,
            )(xpi, spa, wm, cv)
            outs.append(fea)
        else:
            y = pl.pallas_call(
                _last_stage_kernel, grid=_GRID,
                in_specs=[_XSPEC, _SPASPEC, _WMSPEC, _CVSPEC] +
                         [_OSPEC] * 3 + [_W4SPEC, _CVSPEC],
                out_specs=_YSPEC, out_shape=_YSHAPE, compiler_params=_CP,
                scratch_shapes=_SLAB,
            )(xpi, spa, wm, cv, *outs, w4, bcv)
    return y[None], cm
